# baseline (speedup 1.0000x reference)
import functools

import jax
import jax.numpy as jnp
from jax import lax
from jax.experimental import pallas as pl
from jax.experimental.pallas import tpu as pltpu
from jax.experimental.pallas import tpu_sc as plsc

_ROW_BLOCK = 256
_TOKENS = 8192
_KEXP = 64
_NC, _NS = 2, 16
_NW = _NC * _NS
_RPW = _TOKENS // _NW
_BLK = 16


def _logits_body(z_ref, wt_ref, b_ref, out_ref):
    acc = jnp.dot(z_ref[...], wt_ref[...], preferred_element_type=jnp.float32)
    out_ref[...] = acc + b_ref[0:1, :]


@jax.jit
def _logits(z, wt, b2d):
    tokens, dim = z.shape
    kexp = wt.shape[1]
    return pl.pallas_call(
        _logits_body,
        grid=(tokens // _ROW_BLOCK,),
        in_specs=[
            pl.BlockSpec((_ROW_BLOCK, dim), lambda i: (i, 0)),
            pl.BlockSpec((dim, kexp), lambda i: (0, 0)),
            pl.BlockSpec((8, kexp), lambda i: (0, 0)),
        ],
        out_specs=pl.BlockSpec((_ROW_BLOCK, kexp), lambda i: (i, 0)),
        out_shape=jax.ShapeDtypeStruct((tokens, kexp), jnp.float32),
    )(z, wt, b2d)


def _sc_router_body(logits_hbm, out_hbm, lg_v, out_v):
    wid = lax.axis_index("s") * _NC + lax.axis_index("c")
    base_row = wid * _RPW
    pltpu.sync_copy(logits_hbm.at[pl.ds(base_row, _RPW)], lg_v)

    lane = lax.iota(jnp.int32, _BLK)
    zeros16 = jnp.zeros((_BLK,), jnp.float32)

    def block(b, carry):
        rows = b * _BLK + lane
        neg = jnp.full((_BLK,), -3e38, jnp.float32)
        m1, m2 = neg, neg
        i1 = jnp.zeros((_BLK,), jnp.int32)
        i2 = jnp.zeros((_BLK,), jnp.int32)
        for e in range(_KEXP):
            e_v = jnp.full((_BLK,), e, jnp.int32)
            v = plsc.load_gather(lg_v, [rows, e_v])
            gt1 = v > m1
            gt2 = v > m2
            m2 = jnp.where(gt1, m1, jnp.where(gt2, v, m2))
            i2 = jnp.where(gt1, i1, jnp.where(gt2, e_v, i2))
            m1 = jnp.where(gt1, v, m1)
            i1 = jnp.where(gt1, e_v, i1)
        t = jnp.exp(m2 - m1)
        den = 1.0 + t
        w1 = 1.0 / den
        w2 = t / den
        for r in range(_BLK):
            for c in range(_KEXP // _BLK):
                out_v[b * _BLK + r, pl.ds(c * _BLK, _BLK)] = zeros16
        plsc.store_scatter(out_v, [rows, i1], w1)
        plsc.store_scatter(out_v, [rows, i2], w2)
        return carry

    lax.fori_loop(0, _RPW // _BLK, block, 0)
    pltpu.sync_copy(out_v, out_hbm.at[pl.ds(base_row, _RPW)])


@jax.jit
def _sc_router(logits):
    mesh = plsc.VectorSubcoreMesh(
        core_axis_name="c", subcore_axis_name="s",
        num_cores=_NC, num_subcores=_NS,
    )
    return pl.kernel(
        _sc_router_body,
        out_type=jax.ShapeDtypeStruct((_TOKENS, _KEXP), jnp.float32),
        mesh=mesh,
        scratch_types=[
            pltpu.VMEM((_RPW, _KEXP), jnp.float32),
            pltpu.VMEM((_RPW, _KEXP), jnp.float32),
        ],
        compiler_params=pltpu.CompilerParams(
            needs_layout_passes=False,
            use_tc_tiling_on_sc=True,
        ),
    )(logits)


def kernel(z, W, b, k):
    del k
    wt = W.T
    b2d = jnp.broadcast_to(b[None, :], (8, b.shape[0]))
    return _sc_router(_logits(z, wt, b2d))


# speedup vs baseline: 1.2214x; 1.2214x over previous
import functools

import jax
import jax.numpy as jnp
from jax import lax
from jax.experimental import pallas as pl
from jax.experimental.pallas import tpu as pltpu
from jax.experimental.pallas import tpu_sc as plsc

_ROW_BLOCK = 1024
_TOKENS = 8192
_KEXP = 64
_NC, _NS = 2, 16
_NW = _NC * _NS
_RPW = _TOKENS // _NW
_BLK = 16


def _logits_body(z_ref, wt_ref, b_ref, out_ref):
    acc = jnp.dot(z_ref[...], wt_ref[...], preferred_element_type=jnp.float32)
    out_ref[...] = acc + b_ref[0:1, :]


@jax.jit
def _logits(z, wt, b2d):
    tokens, dim = z.shape
    kexp = wt.shape[1]
    return pl.pallas_call(
        _logits_body,
        grid=(tokens // _ROW_BLOCK,),
        in_specs=[
            pl.BlockSpec((_ROW_BLOCK, dim), lambda i: (i, 0)),
            pl.BlockSpec((dim, kexp), lambda i: (0, 0)),
            pl.BlockSpec((8, kexp), lambda i: (0, 0)),
        ],
        out_specs=pl.BlockSpec((_ROW_BLOCK, kexp), lambda i: (i, 0)),
        out_shape=jax.ShapeDtypeStruct((tokens, kexp), jnp.float32),
    )(z, wt, b2d)


def _sc_router_body(logits_hbm, out_hbm, lg_v, out_v):
    wid = lax.axis_index("s") * _NC + lax.axis_index("c")
    base_row = wid * _RPW
    pltpu.sync_copy(logits_hbm.at[pl.ds(base_row, _RPW)], lg_v)

    lane = lax.iota(jnp.int32, _BLK)
    zeros16 = jnp.zeros((_BLK,), jnp.float32)

    def block(b, carry):
        rows = b * _BLK + lane
        neg = jnp.full((_BLK,), -3e38, jnp.float32)
        m1, m2 = neg, neg
        i1 = jnp.zeros((_BLK,), jnp.int32)
        i2 = jnp.zeros((_BLK,), jnp.int32)
        for e in range(_KEXP):
            e_v = jnp.full((_BLK,), e, jnp.int32)
            v = plsc.load_gather(lg_v, [rows, e_v])
            gt1 = v > m1
            gt2 = v > m2
            m2 = jnp.where(gt1, m1, jnp.where(gt2, v, m2))
            i2 = jnp.where(gt1, i1, jnp.where(gt2, e_v, i2))
            m1 = jnp.where(gt1, v, m1)
            i1 = jnp.where(gt1, e_v, i1)
        t = jnp.exp(m2 - m1)
        den = 1.0 + t
        w1 = 1.0 / den
        w2 = t / den
        for r in range(_BLK):
            for c in range(_KEXP // _BLK):
                out_v[b * _BLK + r, pl.ds(c * _BLK, _BLK)] = zeros16
        plsc.store_scatter(out_v, [rows, i1], w1)
        plsc.store_scatter(out_v, [rows, i2], w2)
        return carry

    lax.fori_loop(0, _RPW // _BLK, block, 0)
    pltpu.sync_copy(out_v, out_hbm.at[pl.ds(base_row, _RPW)])


@jax.jit
def _sc_router(logits):
    mesh = plsc.VectorSubcoreMesh(
        core_axis_name="c", subcore_axis_name="s",
        num_cores=_NC, num_subcores=_NS,
    )
    return pl.kernel(
        _sc_router_body,
        out_type=jax.ShapeDtypeStruct((_TOKENS, _KEXP), jnp.float32),
        mesh=mesh,
        scratch_types=[
            pltpu.VMEM((_RPW, _KEXP), jnp.float32),
            pltpu.VMEM((_RPW, _KEXP), jnp.float32),
        ],
        compiler_params=pltpu.CompilerParams(
            needs_layout_passes=False,
            use_tc_tiling_on_sc=True,
        ),
    )(logits)


def kernel(z, W, b, k):
    del k
    wt = W.T
    b2d = jnp.broadcast_to(b[None, :], (8, b.shape[0]))
    return _sc_router(_logits(z, wt, b2d))
